# Initial kernel scaffold; baseline (speedup 1.0000x reference)
import jax
import jax.numpy as jnp
from jax.experimental import pallas as pl


def kernel(batch, X, H, E_idx, E, t, params):
    raise NotImplementedError("write your pallas kernel here")


# speedup vs baseline: 1.0236x; 1.0236x over previous
import functools
import math

import jax
import jax.numpy as jnp
from jax import lax
from jax.experimental import pallas as pl
from jax.experimental.pallas import tpu as pltpu

AF = 64
AC = 16
EC = 5
NL = 2
G = 256
N_NODES = 10000
N_EDGES = 320000

NODE_TILE = 2000
EDGE_TILE = 4000
XP = 16
CW = AF + XP


def _silu(x):
    return x * jax.nn.sigmoid(x)


def _ln(x):
    mu = jnp.mean(x, axis=-1, keepdims=True)
    var = jnp.mean((x - mu) ** 2, axis=-1, keepdims=True)
    return (x - mu) * jax.lax.rsqrt(var + 1e-5)


def _prep_body(hf_ref, bf_ref, t_ref, fr_ref,
               aW_ref, ab_ref, tW1_ref, tb1_ref, tW2_ref, tb2_ref,
               adW_ref, adb_ref, wd_ref, ws_ref,
               h_out, ha_out, hb_out):
    R = hf_ref.shape[0]
    t = t_ref[...]
    fr = fr_ref[...]
    args = t * fr
    te = jnp.concatenate([jnp.cos(args), jnp.sin(args)], axis=-1)
    te = _silu(te @ tW1_ref[...] + tb1_ref[...]) @ tW2_ref[...] + tb2_ref[...]
    ss = te @ adW_ref[...] + adb_ref[...]
    scale = ss[:, :AF]
    shift = ss[:, AF:]
    hf = hf_ref[...]
    oh = (hf == lax.broadcasted_iota(jnp.int32, (R, AC), 1)).astype(jnp.float32)
    H = oh @ aW_ref[...] + ab_ref[...]
    bf = bf_ref[...]
    ohb = (bf == lax.broadcasted_iota(jnp.int32, (R, G), 1)).astype(jnp.float32)
    sc = ohb @ scale
    sh = ohb @ shift
    H = _ln(H) * (1.0 + sc) + sh
    h_out[...] = H
    ha_out[...] = H @ wd_ref[...]
    hb_out[...] = H @ ws_ref[...]


def _run_prep(hf, bf, t2, fr, aW, ab, tW1, tb1, tW2, tb2, adW, adb, wd, ws):
    R = NODE_TILE
    grid = (N_NODES // R,)
    full = lambda s: pl.BlockSpec(s, lambda i: (0, 0))
    return pl.pallas_call(
        _prep_body,
        grid=grid,
        in_specs=[
            pl.BlockSpec((R, 1), lambda i: (i, 0)),
            pl.BlockSpec((R, 1), lambda i: (i, 0)),
            full((G, 1)), full((1, AF // 2)),
            full((AC, AF)), full((1, AF)),
            full((AF, AF)), full((1, AF)),
            full((AF, AF)), full((1, AF)),
            full((AF, 2 * AF)), full((1, 2 * AF)),
            full((AF, AF)), full((AF, AF)),
        ],
        out_specs=[
            pl.BlockSpec((R, AF), lambda i: (i, 0)),
            pl.BlockSpec((R, AF), lambda i: (i, 0)),
            pl.BlockSpec((R, AF), lambda i: (i, 0)),
        ],
        out_shape=[jax.ShapeDtypeStruct((N_NODES, AF), jnp.float32)] * 3,
    )(hf, bf, t2, fr, aW, ab, tW1, tb1, tW2, tb2, adW, adb, wd, ws)


def _edge_body(first, gd_ref, gs_ref, xd_ref, xs_ref, ef_ref, mp_ref,
               etab_ref, wd2_ref, b1_ref, we2_ref, be2_ref,
               wx1_ref, bx1_ref, wx2r_ref, bx2_ref, wem_ref,
               out_ref):
    T = gd_ref.shape[0]
    rel = xd_ref[...] - xs_ref[...]
    d2 = jnp.sum(rel * rel, axis=-1, keepdims=True)
    ef = ef_ref[...]
    ohe = (ef == lax.broadcasted_iota(jnp.int32, (T, 8), 1)).astype(jnp.float32)
    pre = (gd_ref[...] + gs_ref[...] + d2 * wd2_ref[...]
           + ohe @ etab_ref[...] + b1_ref[...])
    if not first:
        pre = pre + mp_ref[...] @ wem_ref[...]
    u = _silu(pre)
    m = _silu(u @ we2_ref[...] + be2_ref[...])
    c1 = _silu(m @ wx1_ref[...] + bx1_ref[...])
    coef = jnp.sum(c1 * wx2r_ref[...], axis=-1, keepdims=True) + bx2_ref[...]
    relc = rel * coef
    if first:
        ones3 = (lax.broadcasted_iota(jnp.int32, (T, XP), 1) == 3)
        relc = relc + ones3.astype(jnp.float32)
    out_ref[...] = jnp.concatenate([m, relc], axis=-1)


def _run_edge(first, gd, gs, xd, xs, ef, mprev,
              etab, wd2, b1, we2, be2, wx1, bx1, wx2r, bx2, wem):
    T = EDGE_TILE
    grid = (N_EDGES // T,)
    full = lambda s: pl.BlockSpec(s, lambda i: tuple(0 for _ in s))
    row = lambda w: pl.BlockSpec((T, w), lambda i: (i, 0))
    return pl.pallas_call(
        functools.partial(_edge_body, first),
        grid=grid,
        in_specs=[
            row(AF), row(AF), row(XP), row(XP), row(1), row(AF),
            full((8, AF)), full((1, AF)), full((1, AF)),
            full((AF, AF)), full((1, AF)),
            full((AF, AF)), full((1, AF)),
            full((1, AF)), full((1, 1)), full((AF, AF)),
        ],
        out_specs=row(CW),
        out_shape=jax.ShapeDtypeStruct((N_EDGES, CW), jnp.float32),
    )(gd, gs, xd, xs, ef, mprev, etab, wd2, b1, we2, be2, wx1, bx1, wx2r, bx2, wem)


def _node0_body(h_ref, xp_ref, p0_ref, p1_ref,
                wh1a_ref, wh1b_ref, bh1_ref, wh2_ref, bh2_ref,
                wdn_ref, wsn_ref, xm_ref,
                h1_out, x1_out, ha_out, hb_out, cnt_out):
    H = h_ref[...]
    S = p0_ref[...] + p1_ref[...]
    agg = S[:, :AF]
    cnt = S[:, AF + 3:AF + 4]
    xs16 = S[:, AF:] * xm_ref[...]
    x1 = xp_ref[...] + xs16 / (cnt + 1.0)
    mid = _silu(H @ wh1a_ref[...] + agg @ wh1b_ref[...] + bh1_ref[...])
    h1 = H + mid @ wh2_ref[...] + bh2_ref[...]
    h1_out[...] = h1
    x1_out[...] = x1
    ha_out[...] = h1 @ wdn_ref[...]
    hb_out[...] = h1 @ wsn_ref[...]
    cnt_out[...] = cnt


def _run_node0(h, xp, p0, p1, wh1a, wh1b, bh1, wh2, bh2, wdn, wsn, xm):
    R = NODE_TILE
    grid = (N_NODES // R,)
    full = lambda s: pl.BlockSpec(s, lambda i: (0, 0))
    row = lambda w: pl.BlockSpec((R, w), lambda i: (i, 0))
    return pl.pallas_call(
        _node0_body,
        grid=grid,
        in_specs=[row(AF), row(XP), row(CW), row(CW),
                  full((AF, AF)), full((AF, AF)), full((1, AF)),
                  full((AF, AF)), full((1, AF)),
                  full((AF, AF)), full((AF, AF)), full((1, XP))],
        out_specs=[row(AF), row(XP), row(AF), row(AF), row(1)],
        out_shape=[
            jax.ShapeDtypeStruct((N_NODES, AF), jnp.float32),
            jax.ShapeDtypeStruct((N_NODES, XP), jnp.float32),
            jax.ShapeDtypeStruct((N_NODES, AF), jnp.float32),
            jax.ShapeDtypeStruct((N_NODES, AF), jnp.float32),
            jax.ShapeDtypeStruct((N_NODES, 1), jnp.float32),
        ],
    )(h, xp, p0, p1, wh1a, wh1b, bh1, wh2, bh2, wdn, wsn, xm)


def _node1_body(h_ref, xp_ref, p0_ref, p1_ref, cnt_ref,
                wh1a_ref, wh1b_ref, bh1_ref, wh2_ref, bh2_ref,
                hw_ref, hb_ref, xm_ref,
                x1_out, hl_out, hp_out):
    H = h_ref[...]
    S = p0_ref[...] + p1_ref[...]
    agg = S[:, :AF]
    cnt = cnt_ref[...]
    xs16 = S[:, AF:] * xm_ref[...]
    x1_out[...] = xp_ref[...] + xs16 / (cnt + 1.0)
    mid = _silu(H @ wh1a_ref[...] + agg @ wh1b_ref[...] + bh1_ref[...])
    h1 = H + mid @ wh2_ref[...] + bh2_ref[...]
    hl = _ln(h1) @ hw_ref[...] + hb_ref[...]
    hl_out[...] = hl
    ex = jnp.exp(hl - jnp.max(hl, axis=-1, keepdims=True))
    hp_out[...] = ex / jnp.sum(ex, axis=-1, keepdims=True)


def _run_node1(h, xp, p0, p1, cnt, wh1a, wh1b, bh1, wh2, bh2, hw, hb, xm):
    R = NODE_TILE
    grid = (N_NODES // R,)
    full = lambda s: pl.BlockSpec(s, lambda i: (0, 0))
    row = lambda w: pl.BlockSpec((R, w), lambda i: (i, 0))
    return pl.pallas_call(
        _node1_body,
        grid=grid,
        in_specs=[row(AF), row(XP), row(CW), row(CW), row(1),
                  full((AF, AF)), full((AF, AF)), full((1, AF)),
                  full((AF, AF)), full((1, AF)),
                  full((AF, AC)), full((1, AC)), full((1, XP))],
        out_specs=[row(XP), row(AC), row(AC)],
        out_shape=[
            jax.ShapeDtypeStruct((N_NODES, XP), jnp.float32),
            jax.ShapeDtypeStruct((N_NODES, AC), jnp.float32),
            jax.ShapeDtypeStruct((N_NODES, AC), jnp.float32),
        ],
    )(h, xp, p0, p1, cnt, wh1a, wh1b, bh1, wh2, bh2, hw, hb, xm)


def _gather_tables(ha, hb, xp, src, dst):
    gd = ha[dst]
    gs = hb[src]
    xd = xp[dst]
    xs = xp[src]
    return gd, gs, xd, xs


def _scatter_edges(eo, dst):
    s = jax.ops.segment_sum(eo, dst, num_segments=N_NODES)
    return s, jnp.zeros_like(s)


def kernel(batch, X, H, E_idx, E, t, params):
    p = params
    f32 = jnp.float32
    src = E_idx[0].astype(jnp.int32)
    dst = E_idx[1].astype(jnp.int32)

    half = AF // 2
    fr = jnp.exp(-math.log(10000.0) * jnp.arange(half, dtype=f32) / half)
    fr = fr.reshape(1, half)
    r1 = lambda v: v.reshape(1, -1).astype(f32)
    lps = [p['layer%d' % l] for l in range(NL)]
    Wd = [lp['We1'][:AF] for lp in lps]
    Ws = [lp['We1'][AF:2 * AF] for lp in lps]
    wd2 = [lp['We1'][2 * AF].reshape(1, AF) for lp in lps]
    WE = [lp['We1'][2 * AF + 1:] for lp in lps]
    etab = []
    b1 = []
    for l, lp in enumerate(lps):
        tab = p['edge_emb_W'] @ WE[l]
        tab = jnp.concatenate([tab, jnp.zeros((8 - EC, AF), f32)], axis=0)
        etab.append(tab)
        bb = lp['be1'] + p['edge_emb_b'] @ WE[l]
        if l == 1:
            bb = bb + lps[0]['bed'] @ WE[l]
        b1.append(bb.reshape(1, AF))
    WEM = [jnp.zeros((AF, AF), f32), lps[0]['Wed'] @ WE[1]]
    wh1a = [lp['Wh1'][:AF] for lp in lps]
    wh1b = [lp['Wh1'][AF:] for lp in lps]
    wx2r = [lp['Wx2'].reshape(1, AF) for lp in lps]
    xmask = (jnp.arange(XP) < 3).astype(f32).reshape(1, XP)

    hf = H.astype(jnp.int32).reshape(N_NODES, 1)
    bf = batch.astype(jnp.int32).reshape(N_NODES, 1)
    ef = E.astype(jnp.int32).reshape(N_EDGES, 1)
    t2 = t.reshape(G, 1)
    xp0 = jnp.concatenate([X, jnp.zeros((N_NODES, XP - 3), f32)], axis=1)

    H0, HA, HB = _run_prep(hf, bf, t2, fr,
                           p['atom_emb_W'], r1(p['atom_emb_b']),
                           p['t_W1'], r1(p['t_b1']), p['t_W2'], r1(p['t_b2']),
                           p['ada_W'], r1(p['ada_b']), Wd[0], Ws[0])

    Hcur, Xcur = H0, xp0
    mprev = jnp.zeros((N_EDGES, AF), f32)
    cnt = None
    for l in range(NL):
        lp = lps[l]
        gd, gs, xd, xs = _gather_tables(HA, HB, Xcur, src, dst)
        eo = _run_edge(l == 0, gd, gs, xd, xs, ef, mprev,
                       etab[l], wd2[l], b1[l],
                       lp['We2'], r1(lp['be2']),
                       lp['Wx1'], r1(lp['bx1']),
                       wx2r[l], lp['bx2'].reshape(1, 1), WEM[l])
        P0, P1 = _scatter_edges(eo, dst)
        if l == 0:
            Hcur, Xcur, HA, HB, cnt = _run_node0(
                Hcur, Xcur, P0, P1,
                wh1a[l], wh1b[l], r1(lp['bh1']), lp['Wh2'], r1(lp['bh2']),
                Wd[1], Ws[1], xmask)
            mprev = eo[:, :AF]
        else:
            Xfin, hl, hp = _run_node1(
                Hcur, Xcur, P0, P1, cnt,
                wh1a[l], wh1b[l], r1(lp['bh1']), lp['Wh2'], r1(lp['bh2']),
                p['head_W'], r1(p['head_b']), xmask)

    x_hat = Xfin[:, :3]
    return (x_hat, hl, hp, hl, hp)


# speedup vs baseline: 4.1071x; 4.0122x over previous
import functools
import math

import jax
import jax.numpy as jnp
from jax import lax
from jax.experimental import pallas as pl
from jax.experimental.pallas import tpu as pltpu
from jax.experimental.pallas import tpu_sc as plsc

AF = 64
AC = 16
EC = 5
NL = 2
G = 256
N_NODES = 10000
N_EDGES = 320000

NODE_TILE = 2000
EDGE_TILE = 4000
XP = 16
CW = 128


def _silu(x):
    return x * jax.nn.sigmoid(x)


def _ln(x):
    mu = jnp.mean(x, axis=-1, keepdims=True)
    var = jnp.mean((x - mu) ** 2, axis=-1, keepdims=True)
    return (x - mu) * jax.lax.rsqrt(var + 1e-5)


def _prep_body(hf_ref, bf_ref, t_ref, fr_ref,
               aW_ref, ab_ref, tW1_ref, tb1_ref, tW2_ref, tb2_ref,
               adW_ref, adb_ref, wd_ref, ws_ref,
               h_out, ha_out, hb_out):
    R = hf_ref.shape[0]
    t = t_ref[...]
    fr = fr_ref[...]
    args = t * fr
    te = jnp.concatenate([jnp.cos(args), jnp.sin(args)], axis=-1)
    te = _silu(te @ tW1_ref[...] + tb1_ref[...]) @ tW2_ref[...] + tb2_ref[...]
    ss = te @ adW_ref[...] + adb_ref[...]
    scale = ss[:, :AF]
    shift = ss[:, AF:]
    hf = hf_ref[...]
    oh = (hf == lax.broadcasted_iota(jnp.int32, (R, AC), 1)).astype(jnp.float32)
    H = oh @ aW_ref[...] + ab_ref[...]
    bf = bf_ref[...]
    ohb = (bf == lax.broadcasted_iota(jnp.int32, (R, G), 1)).astype(jnp.float32)
    sc = ohb @ scale
    sh = ohb @ shift
    H = _ln(H) * (1.0 + sc) + sh
    h_out[...] = H
    ha_out[...] = H @ wd_ref[...]
    hb_out[...] = H @ ws_ref[...]


def _run_prep(hf, bf, t2, fr, aW, ab, tW1, tb1, tW2, tb2, adW, adb, wd, ws):
    R = NODE_TILE
    grid = (N_NODES // R,)
    full = lambda s: pl.BlockSpec(s, lambda i: (0, 0))
    return pl.pallas_call(
        _prep_body,
        grid=grid,
        in_specs=[
            pl.BlockSpec((R, 1), lambda i: (i, 0)),
            pl.BlockSpec((R, 1), lambda i: (i, 0)),
            full((G, 1)), full((1, AF // 2)),
            full((AC, AF)), full((1, AF)),
            full((AF, AF)), full((1, AF)),
            full((AF, AF)), full((1, AF)),
            full((AF, 2 * AF)), full((1, 2 * AF)),
            full((AF, AF)), full((AF, AF)),
        ],
        out_specs=[
            pl.BlockSpec((R, AF), lambda i: (i, 0)),
            pl.BlockSpec((R, AF), lambda i: (i, 0)),
            pl.BlockSpec((R, AF), lambda i: (i, 0)),
        ],
        out_shape=[jax.ShapeDtypeStruct((N_NODES, AF), jnp.float32)] * 3,
    )(hf, bf, t2, fr, aW, ab, tW1, tb1, tW2, tb2, adW, adb, wd, ws)


def _edge_body(first, *refs):
    if first:
        (gd_ref, gs_ref, ef_ref,
         etab_ref, wd2_ref, b1_ref, we2_ref, be2_ref,
         wx1_ref, bx1_ref, wx2r_ref, bx2_ref, out_ref) = refs
    else:
        (gd_ref, gs_ref, ef_ref, mp_ref,
         etab_ref, wd2_ref, b1_ref, we2_ref, be2_ref,
         wx1_ref, bx1_ref, wx2r_ref, bx2_ref, wem_ref, out_ref) = refs
    T = gd_ref.shape[0]
    gd80 = gd_ref[...]
    gs80 = gs_ref[...]
    rel = gd80[:, AF:AF + XP] - gs80[:, AF:AF + XP]
    d2 = jnp.sum(rel * rel, axis=-1, keepdims=True)
    ef = ef_ref[...]
    ohe = (ef == lax.broadcasted_iota(jnp.int32, (T, 8), 1)).astype(jnp.float32)
    pre = (gd80[:, :AF] + gs80[:, :AF] + d2 * wd2_ref[...]
           + ohe @ etab_ref[...] + b1_ref[...])
    if not first:
        pre = pre + mp_ref[...][:, :AF] @ wem_ref[...]
    u = _silu(pre)
    m = _silu(u @ we2_ref[...] + be2_ref[...])
    c1 = _silu(m @ wx1_ref[...] + bx1_ref[...])
    coef = jnp.sum(c1 * wx2r_ref[...], axis=-1, keepdims=True) + bx2_ref[...]
    relc = rel * coef
    if first:
        ones3 = (lax.broadcasted_iota(jnp.int32, (T, XP), 1) == 3)
        relc = relc + ones3.astype(jnp.float32)
    out_ref[...] = jnp.concatenate(
        [m, relc, jnp.zeros((T, CW - AF - XP), jnp.float32)], axis=-1)


def _run_edge(first, gd, gs, ef, mprev,
              etab, wd2, b1, we2, be2, wx1, bx1, wx2r, bx2, wem):
    T = EDGE_TILE
    grid = (N_EDGES // T,)
    full = lambda s: pl.BlockSpec(s, lambda i: tuple(0 for _ in s))
    row = lambda w: pl.BlockSpec((T, w), lambda i: (i, 0))
    wspecs = [full((8, AF)), full((1, AF)), full((1, AF)),
              full((AF, AF)), full((1, AF)),
              full((AF, AF)), full((1, AF)),
              full((1, AF)), full((1, 1))]
    if first:
        ins = [row(CW), row(CW), row(1)] + wspecs
        args = (gd, gs, ef, etab, wd2, b1, we2, be2, wx1, bx1, wx2r, bx2)
    else:
        ins = [row(CW), row(CW), row(1), row(CW)] + wspecs + [full((AF, AF))]
        args = (gd, gs, ef, mprev, etab, wd2, b1, we2, be2, wx1, bx1, wx2r,
                bx2, wem)
    return pl.pallas_call(
        functools.partial(_edge_body, first),
        grid=grid,
        in_specs=ins,
        out_specs=row(CW),
        out_shape=jax.ShapeDtypeStruct((N_EDGES, CW), jnp.float32),
    )(*args)


def _node0_body(h_ref, xp_ref, p0_ref, p1_ref,
                wh1a_ref, wh1b_ref, bh1_ref, wh2_ref, bh2_ref,
                wdn_ref, wsn_ref, xm_ref,
                h1_out, x1_out, ha_out, hb_out, cnt_out):
    H = h_ref[...]
    S = p0_ref[...] + p1_ref[...]
    agg = S[:, :AF]
    cnt = S[:, AF + 3:AF + 4]
    xs16 = S[:, AF:AF + XP] * xm_ref[...]
    x1 = xp_ref[...] + xs16 / (cnt + 1.0)
    mid = _silu(H @ wh1a_ref[...] + agg @ wh1b_ref[...] + bh1_ref[...])
    h1 = H + mid @ wh2_ref[...] + bh2_ref[...]
    h1_out[...] = h1
    x1_out[...] = x1
    ha_out[...] = h1 @ wdn_ref[...]
    hb_out[...] = h1 @ wsn_ref[...]
    cnt_out[...] = cnt


def _run_node0(h, xp, p0, p1, wh1a, wh1b, bh1, wh2, bh2, wdn, wsn, xm):
    R = NODE_TILE
    grid = (N_NODES // R,)
    full = lambda s: pl.BlockSpec(s, lambda i: (0, 0))
    row = lambda w: pl.BlockSpec((R, w), lambda i: (i, 0))
    return pl.pallas_call(
        _node0_body,
        grid=grid,
        in_specs=[row(AF), row(XP), row(CW), row(CW),
                  full((AF, AF)), full((AF, AF)), full((1, AF)),
                  full((AF, AF)), full((1, AF)),
                  full((AF, AF)), full((AF, AF)), full((1, XP))],
        out_specs=[row(AF), row(XP), row(AF), row(AF), row(1)],
        out_shape=[
            jax.ShapeDtypeStruct((N_NODES, AF), jnp.float32),
            jax.ShapeDtypeStruct((N_NODES, XP), jnp.float32),
            jax.ShapeDtypeStruct((N_NODES, AF), jnp.float32),
            jax.ShapeDtypeStruct((N_NODES, AF), jnp.float32),
            jax.ShapeDtypeStruct((N_NODES, 1), jnp.float32),
        ],
    )(h, xp, p0, p1, wh1a, wh1b, bh1, wh2, bh2, wdn, wsn, xm)


def _node1_body(h_ref, xp_ref, p0_ref, p1_ref, cnt_ref,
                wh1a_ref, wh1b_ref, bh1_ref, wh2_ref, bh2_ref,
                hw_ref, hb_ref, xm_ref,
                x1_out, hl_out, hp_out):
    H = h_ref[...]
    S = p0_ref[...] + p1_ref[...]
    agg = S[:, :AF]
    cnt = cnt_ref[...]
    xs16 = S[:, AF:AF + XP] * xm_ref[...]
    x1_out[...] = xp_ref[...] + xs16 / (cnt + 1.0)
    mid = _silu(H @ wh1a_ref[...] + agg @ wh1b_ref[...] + bh1_ref[...])
    h1 = H + mid @ wh2_ref[...] + bh2_ref[...]
    hl = _ln(h1) @ hw_ref[...] + hb_ref[...]
    hl_out[...] = hl
    ex = jnp.exp(hl - jnp.max(hl, axis=-1, keepdims=True))
    hp_out[...] = ex / jnp.sum(ex, axis=-1, keepdims=True)


def _run_node1(h, xp, p0, p1, cnt, wh1a, wh1b, bh1, wh2, bh2, hw, hb, xm):
    R = NODE_TILE
    grid = (N_NODES // R,)
    full = lambda s: pl.BlockSpec(s, lambda i: (0, 0))
    row = lambda w: pl.BlockSpec((R, w), lambda i: (i, 0))
    return pl.pallas_call(
        _node1_body,
        grid=grid,
        in_specs=[row(AF), row(XP), row(CW), row(CW), row(1),
                  full((AF, AF)), full((AF, AF)), full((1, AF)),
                  full((AF, AF)), full((1, AF)),
                  full((AF, AC)), full((1, AC)), full((1, XP))],
        out_specs=[row(XP), row(AC), row(AC)],
        out_shape=[
            jax.ShapeDtypeStruct((N_NODES, XP), jnp.float32),
            jax.ShapeDtypeStruct((N_NODES, AC), jnp.float32),
            jax.ShapeDtypeStruct((N_NODES, AC), jnp.float32),
        ],
    )(h, xp, p0, p1, cnt, wh1a, wh1b, bh1, wh2, bh2, hw, hb, xm)


NW = 32
E_PER_W = N_EDGES // NW
CHUNK = 128
NCH = E_PER_W // CHUNK
TAIL = E_PER_W - NCH * CHUNK
ACC_ROWS = 10240
ZR = ACC_ROWS // 16


def _gather_body(td_hbm, ts_hbm, dst_hbm, src_hbm, gd_hbm, gs_hbm,
                 idxd, idxs, rowsd, rowss, idxdt, idxst, rowsdt, rowsst,
                 semd, sems):
    wid = lax.axis_index("s") * 2 + lax.axis_index("c")
    base = wid * E_PER_W

    def body(i, carry):
        off = base + i * CHUNK
        pltpu.sync_copy(dst_hbm.at[pl.ds(off, CHUNK)], idxd)
        pltpu.sync_copy(src_hbm.at[pl.ds(off, CHUNK)], idxs)
        cd = pltpu.async_copy(td_hbm.at[idxd], rowsd, semd)
        cs = pltpu.async_copy(ts_hbm.at[idxs], rowss, sems)
        cd.wait()
        cs.wait()
        pltpu.sync_copy(rowsd, gd_hbm.at[pl.ds(off, CHUNK)])
        pltpu.sync_copy(rowss, gs_hbm.at[pl.ds(off, CHUNK)])
        return carry

    lax.fori_loop(0, NCH, body, 0)
    off = base + NCH * CHUNK
    pltpu.sync_copy(dst_hbm.at[pl.ds(off, TAIL)], idxdt)
    pltpu.sync_copy(src_hbm.at[pl.ds(off, TAIL)], idxst)
    cd = pltpu.async_copy(td_hbm.at[idxdt], rowsdt, semd)
    cs = pltpu.async_copy(ts_hbm.at[idxst], rowsst, sems)
    cd.wait()
    cs.wait()
    pltpu.sync_copy(rowsdt, gd_hbm.at[pl.ds(off, TAIL)])
    pltpu.sync_copy(rowsst, gs_hbm.at[pl.ds(off, TAIL)])


def _sc_gather(td, ts, dst, src):
    f32 = jnp.float32
    mesh = plsc.VectorSubcoreMesh(core_axis_name="c", subcore_axis_name="s")
    fn = pl.kernel(
        _gather_body,
        out_type=[jax.ShapeDtypeStruct((N_EDGES, CW), f32)] * 2,
        mesh=mesh,
        scratch_types=[
            pltpu.VMEM((CHUNK,), jnp.int32),
            pltpu.VMEM((CHUNK,), jnp.int32),
            pltpu.VMEM((CHUNK, CW), f32),
            pltpu.VMEM((CHUNK, CW), f32),
            pltpu.VMEM((TAIL,), jnp.int32),
            pltpu.VMEM((TAIL,), jnp.int32),
            pltpu.VMEM((TAIL, CW), f32),
            pltpu.VMEM((TAIL, CW), f32),
            pltpu.SemaphoreType.DMA,
            pltpu.SemaphoreType.DMA,
        ],
    )
    return fn(td, ts, dst, src)


def _scatter_body(eo_hbm, dst_hbm, z_hbm, out_hbm,
                  acc, idx, rows, idxt, rowst, sem):
    c = lax.axis_index("c")
    s = lax.axis_index("s")
    wid = s * 2 + c
    base = wid * E_PER_W
    pltpu.sync_copy(z_hbm, acc.at[pl.ds(s * ZR, ZR)])
    plsc.subcore_barrier()

    def body(i, carry):
        off = base + i * CHUNK
        pltpu.sync_copy(dst_hbm.at[pl.ds(off, CHUNK)], idx)
        pltpu.sync_copy(eo_hbm.at[pl.ds(off, CHUNK)], rows)
        pltpu.sync_copy(rows, acc.at[idx], add=True)
        return carry

    lax.fori_loop(0, NCH, body, 0)
    off = base + NCH * CHUNK
    pltpu.sync_copy(dst_hbm.at[pl.ds(off, TAIL)], idxt)
    pltpu.sync_copy(eo_hbm.at[pl.ds(off, TAIL)], rowst)
    pltpu.sync_copy(rowst, acc.at[idxt], add=True)
    plsc.subcore_barrier()
    pltpu.sync_copy(acc.at[pl.ds(s * ZR, ZR)], out_hbm.at[c, pl.ds(s * ZR, ZR)])


def _sc_scatter(eo, dst):
    f32 = jnp.float32
    mesh = plsc.VectorSubcoreMesh(core_axis_name="c", subcore_axis_name="s")
    z = jnp.zeros((ZR, CW), f32)
    fn = pl.kernel(
        _scatter_body,
        out_type=jax.ShapeDtypeStruct((2, ACC_ROWS, CW), f32),
        mesh=mesh,
        scratch_types=[
            pltpu.VMEM_SHARED((ACC_ROWS, CW), f32),
            pltpu.VMEM((CHUNK,), jnp.int32),
            pltpu.VMEM((CHUNK, CW), f32),
            pltpu.VMEM((TAIL,), jnp.int32),
            pltpu.VMEM((TAIL, CW), f32),
            pltpu.SemaphoreType.DMA,
        ],
    )
    return fn(eo, dst, z)


def kernel(batch, X, H, E_idx, E, t, params):
    p = params
    f32 = jnp.float32
    src = E_idx[0].astype(jnp.int32)
    dst = E_idx[1].astype(jnp.int32)

    half = AF // 2
    fr = jnp.exp(-math.log(10000.0) * jnp.arange(half, dtype=f32) / half)
    fr = fr.reshape(1, half)
    r1 = lambda v: v.reshape(1, -1).astype(f32)
    lps = [p['layer%d' % l] for l in range(NL)]
    Wd = [lp['We1'][:AF] for lp in lps]
    Ws = [lp['We1'][AF:2 * AF] for lp in lps]
    wd2 = [lp['We1'][2 * AF].reshape(1, AF) for lp in lps]
    WE = [lp['We1'][2 * AF + 1:] for lp in lps]
    etab = []
    b1 = []
    for l, lp in enumerate(lps):
        tab = p['edge_emb_W'] @ WE[l]
        tab = jnp.concatenate([tab, jnp.zeros((8 - EC, AF), f32)], axis=0)
        etab.append(tab)
        bb = lp['be1'] + p['edge_emb_b'] @ WE[l]
        if l == 1:
            bb = bb + lps[0]['bed'] @ WE[l]
        b1.append(bb.reshape(1, AF))
    WEM = [jnp.zeros((AF, AF), f32), lps[0]['Wed'] @ WE[1]]
    wh1a = [lp['Wh1'][:AF] for lp in lps]
    wh1b = [lp['Wh1'][AF:] for lp in lps]
    wx2r = [lp['Wx2'].reshape(1, AF) for lp in lps]
    xmask = (jnp.arange(XP) < 3).astype(f32).reshape(1, XP)

    hf = H.astype(jnp.int32).reshape(N_NODES, 1)
    bf = batch.astype(jnp.int32).reshape(N_NODES, 1)
    ef = E.astype(jnp.int32).reshape(N_EDGES, 1)
    t2 = t.reshape(G, 1)
    xp0 = jnp.concatenate([X, jnp.zeros((N_NODES, XP - 3), f32)], axis=1)

    H0, HA, HB = _run_prep(hf, bf, t2, fr,
                           p['atom_emb_W'], r1(p['atom_emb_b']),
                           p['t_W1'], r1(p['t_b1']), p['t_W2'], r1(p['t_b2']),
                           p['ada_W'], r1(p['ada_b']), Wd[0], Ws[0])

    Hcur, Xcur = H0, xp0
    mprev = None
    cnt = None
    for l in range(NL):
        lp = lps[l]
        zp = jnp.zeros((N_NODES, CW - AF - XP), f32)
        td = jnp.concatenate([HA, Xcur, zp], axis=1)
        ts = jnp.concatenate([HB, Xcur, zp], axis=1)
        gd, gs = _sc_gather(td, ts, dst, src)
        eo = _run_edge(l == 0, gd, gs, ef, mprev,
                       etab[l], wd2[l], b1[l],
                       lp['We2'], r1(lp['be2']),
                       lp['Wx1'], r1(lp['bx1']),
                       wx2r[l], lp['bx2'].reshape(1, 1), WEM[l])
        S = _sc_scatter(eo, dst)
        P0 = S[0, :N_NODES]
        P1 = S[1, :N_NODES]
        if l == 0:
            Hcur, Xcur, HA, HB, cnt = _run_node0(
                Hcur, Xcur, P0, P1,
                wh1a[l], wh1b[l], r1(lp['bh1']), lp['Wh2'], r1(lp['bh2']),
                Wd[1], Ws[1], xmask)
            mprev = eo
        else:
            Xfin, hl, hp = _run_node1(
                Hcur, Xcur, P0, P1, cnt,
                wh1a[l], wh1b[l], r1(lp['bh1']), lp['Wh2'], r1(lp['bh2']),
                p['head_W'], r1(p['head_b']), xmask)

    x_hat = Xfin[:, :3]
    return (x_hat, hl, hp, hl, hp)
